# baseline (speedup 1.0000x reference)
import functools

import jax
import jax.numpy as jnp
from jax import lax
from jax.experimental import pallas as pl
from jax.experimental.pallas import tpu as pltpu
from jax.experimental.pallas import tpu_sc as plsc

DIM = 128
NUM_CORES = 2
NUM_SUBCORES = 16
NUM_WORKERS = NUM_CORES * NUM_SUBCORES
CHUNK = 224
NBUF = 4


def kernel(x, weight):
    b0, b1 = x.shape
    b1p = (b1 + 7) // 8 * 8
    xp = jnp.pad(x.astype(jnp.int32), ((0, 0), (0, b1p - b1)))
    idx = xp.reshape(b0 * b1p)
    num_idx = b0 * b1p
    per_worker = num_idx // NUM_WORKERS
    n_chunks = per_worker // CHUNK
    n_groups = n_chunks // NBUF

    mesh = plsc.VectorSubcoreMesh(core_axis_name="c", subcore_axis_name="s")

    @functools.partial(
        pl.kernel,
        mesh=mesh,
        out_type=jax.ShapeDtypeStruct((num_idx, DIM), jnp.float32),
        scratch_types=[
            pltpu.VMEM((per_worker,), jnp.int32),
            pltpu.VMEM((NBUF, CHUNK, DIM), jnp.float32),
            pltpu.SemaphoreType.DMA((NBUF,)),
        ],
    )
    def gather_kernel(table_hbm, idx_hbm, out_hbm, idx_v, rows_v, sems):
        wid = lax.axis_index("s") * NUM_CORES + lax.axis_index("c")
        base = wid * per_worker

        def gather_desc(i, b):
            return pltpu.make_async_copy(
                table_hbm.at[idx_v.at[pl.ds(i * CHUNK, CHUNK)]],
                rows_v.at[b],
                sems.at[b],
            )

        def store(i, b):
            pltpu.sync_copy(rows_v.at[b], out_hbm.at[pl.ds(base + i * CHUNK, CHUNK)])

        pltpu.sync_copy(idx_hbm.at[pl.ds(base, per_worker)], idx_v)

        for b in range(NBUF):
            gather_desc(b, b).start()

        @pl.loop(0, n_groups - 1)
        def _(g):
            for b in range(NBUF):
                i = g * NBUF + b
                gather_desc(i, b).wait()
                store(i, b)
                gather_desc(i + NBUF, b).start()

        for b in range(NBUF):
            i = (n_groups - 1) * NBUF + b
            gather_desc(i, b).wait()
            store(i, b)

    out = gather_kernel(weight, idx)
    return out.reshape(b0, b1p, DIM)[:, :b1, :]


# speedup vs baseline: 4.3275x; 4.3275x over previous
import functools

import jax
import jax.numpy as jnp
from jax import lax
from jax.experimental import pallas as pl
from jax.experimental.pallas import tpu as pltpu
from jax.experimental.pallas import tpu_sc as plsc

DIM = 128
NUM_CORES = 2
NUM_SUBCORES = 16
NUM_WORKERS = NUM_CORES * NUM_SUBCORES
NE = 4
NBUF = 4
NPARTS = 4


def kernel(x, weight):
    b0, b1 = x.shape
    num_idx = b0 * b1
    idx = x.reshape(num_idx).astype(jnp.int32)
    pb0 = b0 // NPARTS
    chunk = NE * b1
    per_worker = pb0 * b1 // NUM_WORKERS
    elems_per_worker = pb0 // NUM_WORKERS
    n_chunks = elems_per_worker // NE
    n_groups = n_chunks // NBUF

    mesh = plsc.VectorSubcoreMesh(core_axis_name="c", subcore_axis_name="s")

    @functools.partial(
        pl.kernel,
        mesh=mesh,
        out_type=jax.ShapeDtypeStruct((pb0, b1, DIM), jnp.float32),
        scratch_types=[
            pltpu.VMEM((per_worker,), jnp.int32),
            pltpu.VMEM((NBUF, chunk, DIM), jnp.float32),
            pltpu.SemaphoreType.DMA((NBUF,)),
        ],
    )
    def gather_kernel(table_hbm, idx_hbm, out_hbm, idx_v, rows_v, sems):
        wid = lax.axis_index("s") * NUM_CORES + lax.axis_index("c")
        base = wid * per_worker
        ebase = wid * elems_per_worker

        def gather_desc(i, b):
            return pltpu.make_async_copy(
                table_hbm.at[idx_v.at[pl.ds(i * chunk, chunk)]],
                rows_v.at[b],
                sems.at[b],
            )

        def store(i, b):
            for j in range(NE):
                pltpu.sync_copy(
                    rows_v.at[b, pl.ds(j * b1, b1)],
                    out_hbm.at[ebase + i * NE + j],
                )

        pltpu.sync_copy(idx_hbm.at[pl.ds(base, per_worker)], idx_v)

        for b in range(NBUF):
            gather_desc(b, b).start()

        @pl.loop(0, n_groups - 1)
        def _(g):
            for b in range(NBUF):
                i = g * NBUF + b
                gather_desc(i, b).wait()
                store(i, b)
                gather_desc(i + NBUF, b).start()

        for b in range(NBUF):
            i = (n_groups - 1) * NBUF + b
            gather_desc(i, b).wait()
            store(i, b)

    parts = [
        gather_kernel(weight, lax.dynamic_slice_in_dim(idx, p * pb0 * b1, pb0 * b1))
        for p in range(NPARTS)
    ]
    return jnp.concatenate(parts, axis=0)


# speedup vs baseline: 13.8817x; 3.2078x over previous
import functools

import jax
import jax.numpy as jnp
from jax import lax
from jax.experimental import pallas as pl
from jax.experimental.pallas import tpu as pltpu
from jax.experimental.pallas import tpu_sc as plsc

DIM = 128
NUM_CORES = 2
NUM_SUBCORES = 16
NUM_WORKERS = NUM_CORES * NUM_SUBCORES
CHUNK = 400
NBUF = 2


def kernel(x, weight):
    b0, b1 = x.shape
    num_idx = b0 * b1
    idx = x.astype(jnp.int32).T.reshape(num_idx)
    per_worker = num_idx // NUM_WORKERS
    n_chunks = per_worker // CHUNK
    n_groups = n_chunks // NBUF

    mesh = plsc.VectorSubcoreMesh(core_axis_name="c", subcore_axis_name="s")

    @functools.partial(
        pl.kernel,
        mesh=mesh,
        out_type=jax.ShapeDtypeStruct((num_idx, DIM), jnp.float32),
        scratch_types=[
            pltpu.VMEM((per_worker,), jnp.int32),
            pltpu.VMEM((NBUF, CHUNK, DIM), jnp.float32),
            pltpu.SemaphoreType.DMA((NBUF,)),
        ],
    )
    def gather_kernel(table_hbm, idx_hbm, out_hbm, idx_v, rows_v, sems):
        wid = lax.axis_index("s") * NUM_CORES + lax.axis_index("c")
        base = wid * per_worker

        def gather_desc(i, b):
            return pltpu.make_async_copy(
                table_hbm.at[idx_v.at[pl.ds(i * CHUNK, CHUNK)]],
                rows_v.at[b],
                sems.at[b],
            )

        def store(i, b):
            pltpu.sync_copy(rows_v.at[b], out_hbm.at[pl.ds(base + i * CHUNK, CHUNK)])

        pltpu.sync_copy(idx_hbm.at[pl.ds(base, per_worker)], idx_v)

        for b in range(NBUF):
            gather_desc(b, b).start()

        @pl.loop(0, n_groups - 1)
        def _(g):
            for b in range(NBUF):
                i = g * NBUF + b
                gather_desc(i, b).wait()
                store(i, b)
                gather_desc(i + NBUF, b).start()

        for b in range(NBUF):
            i = (n_groups - 1) * NBUF + b
            gather_desc(i, b).wait()
            store(i, b)

    out = gather_kernel(weight, idx)
    return out.reshape(b1, b0, DIM).transpose(1, 0, 2)


# speedup vs baseline: 13.9015x; 1.0014x over previous
import functools

import jax
import jax.numpy as jnp
from jax import lax
from jax.experimental import pallas as pl
from jax.experimental.pallas import tpu as pltpu
from jax.experimental.pallas import tpu_sc as plsc

DIM = 128
NUM_CORES = 2
NUM_SUBCORES = 16
NUM_WORKERS = NUM_CORES * NUM_SUBCORES
CHUNK = 200
NBUF = 4


def kernel(x, weight):
    b0, b1 = x.shape
    num_idx = b0 * b1
    idx = x.astype(jnp.int32).T.reshape(num_idx)
    per_worker = num_idx // NUM_WORKERS
    n_chunks = per_worker // CHUNK
    n_groups = n_chunks // NBUF

    mesh = plsc.VectorSubcoreMesh(core_axis_name="c", subcore_axis_name="s")

    @functools.partial(
        pl.kernel,
        mesh=mesh,
        out_type=jax.ShapeDtypeStruct((num_idx, DIM), jnp.float32),
        scratch_types=[
            pltpu.VMEM((per_worker,), jnp.int32),
            pltpu.VMEM((NBUF, CHUNK, DIM), jnp.float32),
            pltpu.SemaphoreType.DMA((NBUF,)),
            pltpu.SemaphoreType.DMA((NBUF,)),
        ],
    )
    def gather_kernel(table_hbm, idx_hbm, out_hbm, idx_v, rows_v, gsems, ssems):
        wid = lax.axis_index("s") * NUM_CORES + lax.axis_index("c")
        base = wid * per_worker

        def gather_desc(i, b):
            return pltpu.make_async_copy(
                table_hbm.at[idx_v.at[pl.ds(i * CHUNK, CHUNK)]],
                rows_v.at[b],
                gsems.at[b],
            )

        def store_desc(i, b):
            return pltpu.make_async_copy(
                rows_v.at[b],
                out_hbm.at[pl.ds(base + i * CHUNK, CHUNK)],
                ssems.at[b],
            )

        pltpu.sync_copy(idx_hbm.at[pl.ds(base, per_worker)], idx_v)

        gather_desc(0, 0).start()
        gather_desc(1, 1).start()

        def step(i, b, refill, drain):
            if drain:
                store_desc(i - 2, (b + 2) % NBUF).wait()
            if refill:
                gather_desc(i + 2, (b + 2) % NBUF).start()
            gather_desc(i, b).wait()
            store_desc(i, b).start()

        for i in range(NBUF):
            step(i, i, refill=True, drain=i >= 2)

        @pl.loop(1, n_groups - 1)
        def _(g):
            for b in range(NBUF):
                step(g * NBUF + b, b, refill=True, drain=True)

        last = (n_groups - 1) * NBUF
        for b in range(NBUF):
            step(last + b, b, refill=b < 2, drain=True)
        store_desc(last + 2, 2).wait()
        store_desc(last + 3, 3).wait()

    out = gather_kernel(weight, idx)
    return out.reshape(b1, b0, DIM).transpose(1, 0, 2)
